# baseline (speedup 1.0000x reference)
import functools

import jax
import jax.numpy as jnp
from jax import lax
from jax.experimental import pallas as pl
from jax.experimental.pallas import tpu as pltpu
from jax.experimental.pallas import tpu_sc as plsc

_NW = 32


_SUP = 8


def _sc_gather(idx2d, table):
    nrows = idx2d.shape[0]
    d = table.shape[1]
    rows_w = nrows // _NW
    chb = 2 * _SUP
    nouter = rows_w // chb
    assert rows_w * _NW == nrows and nouter * chb == rows_w
    mesh = plsc.VectorSubcoreMesh(core_axis_name="c", subcore_axis_name="s")

    @functools.partial(
        pl.kernel,
        mesh=mesh,
        out_type=jax.ShapeDtypeStruct((nrows, 128, d), table.dtype),
        scratch_types=[
            pltpu.VMEM((chb, 128), jnp.int32),
            pltpu.VMEM((128, d), table.dtype),
            pltpu.VMEM((128, d), table.dtype),
            pltpu.SemaphoreType.DMA,
        ],
    )
    def k(idx_hbm, table_hbm, out_hbm, idx_v, rows_a, rows_b, gsem):
        bufs = (rows_a, rows_b)
        w = lax.axis_index("s") * 2 + lax.axis_index("c")
        start = rows_w * w

        def body(o, carry):
            cb = start + o * chb
            pltpu.sync_copy(idx_hbm.at[pl.ds(cb, chb)], idx_v)
            h = pltpu.async_copy(table_hbm.at[idx_v.at[0]], bufs[0], gsem)
            for t in range(chb):
                p = t % 2
                h.wait()
                if t < chb - 1:
                    h = pltpu.async_copy(
                        table_hbm.at[idx_v.at[t + 1]], bufs[1 - p], gsem
                    )
                pltpu.sync_copy(bufs[p], out_hbm.at[cb + t])
            return carry

        lax.fori_loop(0, nouter, body, 0)

    return k(idx2d, table)


def _tc_linear(fflat, Wt, b2, E, tile):
    nb = E // tile
    out_f = Wt.shape[2]
    F = Wt.shape[1]

    def body(f0, f1, f2, f3, f4, wref, bref, oref):
        a0, a1, a2, a3, a4 = f0[...], f1[...], f2[...], f3[...], f4[...]
        cs = (a0, a1 + a3, a2 + a4, jnp.abs(a1 - a3), jnp.abs(a2 - a4))
        acc = jnp.broadcast_to(bref[...].astype(jnp.float32), (tile, out_f))
        for i, c in enumerate(cs):
            acc += jnp.dot(
                c.astype(jnp.bfloat16), wref[i], preferred_element_type=jnp.float32
            )
        oref[...] = acc

    fspec = [
        pl.BlockSpec((tile, F), lambda i, k=k: (k * nb + i, 0)) for k in range(5)
    ]
    return pl.pallas_call(
        body,
        grid=(nb,),
        in_specs=fspec
        + [
            pl.BlockSpec((5, F, out_f), lambda i: (0, 0, 0)),
            pl.BlockSpec((1, out_f), lambda i: (0, 0)),
        ],
        out_specs=pl.BlockSpec((tile, out_f), lambda i: (i, 0)),
        out_shape=jax.ShapeDtypeStruct((E, out_f), jnp.float32),
    )(fflat, fflat, fflat, fflat, fflat, Wt, b2)


def kernel(x, edgemat, W, b):
    _, F, E, _ = x.shape
    K = edgemat.shape[2]
    out_f = W.shape[0]
    xt = jnp.transpose(x[0, :, :, 0])
    idx2d = jnp.transpose(edgemat[0]).reshape(-1, 128)
    nrows = idx2d.shape[0]
    align = _NW * 2 * _SUP
    npad = (nrows + align - 1) // align * align
    idx_pad = jnp.concatenate(
        [idx2d, jnp.zeros((npad - nrows, 128), jnp.int32)], axis=0
    )
    fgath = _sc_gather(idx_pad, xt)
    Wt = jnp.transpose(W.reshape(out_f, K, F), (1, 2, 0)).astype(jnp.bfloat16)
    y = _tc_linear(fgath.reshape(npad * 128, F), Wt, b.reshape(1, -1), E, tile=512)
    return jnp.transpose(y)[None, :, :, None]


# speedup vs baseline: 1.1962x; 1.1962x over previous
import functools

import jax
import jax.numpy as jnp
from jax import lax
from jax.experimental import pallas as pl
from jax.experimental.pallas import tpu as pltpu
from jax.experimental.pallas import tpu_sc as plsc

_NW = 32


_SUP = 8


def _sc_gather(idx2d, table):
    nrows = idx2d.shape[0]
    d = table.shape[1]
    rows_w = nrows // _NW
    chb = 2 * _SUP
    nouter = rows_w // chb
    assert rows_w * _NW == nrows and nouter * chb == rows_w
    mesh = plsc.VectorSubcoreMesh(core_axis_name="c", subcore_axis_name="s")

    @functools.partial(
        pl.kernel,
        mesh=mesh,
        out_type=jax.ShapeDtypeStruct((nrows, 128, d), table.dtype),
        scratch_types=[
            pltpu.VMEM((128,), jnp.int32),
            pltpu.VMEM((128,), jnp.int32),
            pltpu.VMEM((128, d), table.dtype),
            pltpu.VMEM((128, d), table.dtype),
            pltpu.SemaphoreType.DMA,
            pltpu.SemaphoreType.DMA,
        ],
    )
    def k(idx_hbm, table_hbm, out_hbm, ia, ib, rows_a, rows_b, gsem, isem):
        ibufs = (ia, ib)
        bufs = (rows_a, rows_b)
        w = lax.axis_index("s") * 2 + lax.axis_index("c")

        def body(o, carry):
            def c_of(t):
                return w + _NW * (o * chb + t)

            pltpu.sync_copy(idx_hbm.at[c_of(0)], ia)
            h_i = pltpu.async_copy(idx_hbm.at[c_of(1)], ib, isem)
            h_g = pltpu.async_copy(table_hbm.at[ia], bufs[0], gsem)
            for t in range(chb):
                p = t % 2
                h_g.wait()
                if t < chb - 1:
                    h_i.wait()
                    h_g = pltpu.async_copy(
                        table_hbm.at[ibufs[(t + 1) % 2]], bufs[1 - p], gsem
                    )
                    if t < chb - 2:
                        h_i = pltpu.async_copy(idx_hbm.at[c_of(t + 2)], ibufs[p], isem)
                pltpu.sync_copy(bufs[p], out_hbm.at[c_of(t)])
            return carry

        lax.fori_loop(0, nouter, body, 0)

    return k(idx2d, table)


def _tc_linear(fflat, Wt, b2, E, tile):
    nb = E // tile
    out_f = Wt.shape[2]
    F = Wt.shape[1]

    def body(f0, f1, f2, f3, f4, wref, bref, oref):
        a0, a1, a2, a3, a4 = f0[...], f1[...], f2[...], f3[...], f4[...]
        cs = (a0, a1 + a3, a2 + a4, jnp.abs(a1 - a3), jnp.abs(a2 - a4))
        acc = jnp.broadcast_to(bref[...].astype(jnp.float32), (tile, out_f))
        for i, c in enumerate(cs):
            acc += jnp.dot(
                c.astype(jnp.bfloat16), wref[i], preferred_element_type=jnp.float32
            )
        oref[...] = acc

    fspec = [
        pl.BlockSpec((tile, F), lambda i, k=k: (k * nb + i, 0)) for k in range(5)
    ]
    return pl.pallas_call(
        body,
        grid=(nb,),
        in_specs=fspec
        + [
            pl.BlockSpec((5, F, out_f), lambda i: (0, 0, 0)),
            pl.BlockSpec((1, out_f), lambda i: (0, 0)),
        ],
        out_specs=pl.BlockSpec((tile, out_f), lambda i: (i, 0)),
        out_shape=jax.ShapeDtypeStruct((E, out_f), jnp.float32),
    )(fflat, fflat, fflat, fflat, fflat, Wt, b2)


def kernel(x, edgemat, W, b):
    _, F, E, _ = x.shape
    K = edgemat.shape[2]
    out_f = W.shape[0]
    xt = jnp.transpose(x[0, :, :, 0])
    idx2d = jnp.transpose(edgemat[0]).reshape(-1, 128)
    nrows = idx2d.shape[0]
    align = _NW * 2 * _SUP
    npad = (nrows + align - 1) // align * align
    idx_pad = jnp.concatenate(
        [idx2d, jnp.zeros((npad - nrows, 128), jnp.int32)], axis=0
    )
    fgath = _sc_gather(idx_pad, xt)
    Wt = jnp.transpose(W.reshape(out_f, K, F), (1, 2, 0)).astype(jnp.bfloat16)
    y = _tc_linear(fgath.reshape(npad * 128, F), Wt, b.reshape(1, -1), E, tile=512)
    return jnp.transpose(y)[None, :, :, None]


# speedup vs baseline: 1.8952x; 1.5844x over previous
import functools

import jax
import jax.numpy as jnp
from jax import lax
from jax.experimental import pallas as pl
from jax.experimental.pallas import tpu as pltpu
from jax.experimental.pallas import tpu_sc as plsc

_NW = 32


_SUP = 1


def _sc_gather(idx2d, table):
    nrows = idx2d.shape[0]
    d = table.shape[1]
    rows_w = nrows // _NW
    chb = 2 * _SUP
    nouter = rows_w // chb
    assert rows_w * _NW == nrows and nouter * chb == rows_w
    mesh = plsc.VectorSubcoreMesh(core_axis_name="c", subcore_axis_name="s")

    @functools.partial(
        pl.kernel,
        mesh=mesh,
        out_type=jax.ShapeDtypeStruct((nrows, 128, d), table.dtype),
        scratch_types=[
            pltpu.VMEM((128,), jnp.int32),
            pltpu.VMEM((128,), jnp.int32),
            pltpu.VMEM((128, d), table.dtype),
            pltpu.VMEM((128, d), table.dtype),
            pltpu.SemaphoreType.DMA,
            pltpu.SemaphoreType.DMA,
        ],
    )
    def k(idx_hbm, table_hbm, out_hbm, ia, ib, rows_a, rows_b, gsem, isem):
        ibufs = (ia, ib)
        bufs = (rows_a, rows_b)
        w = lax.axis_index("s") * 2 + lax.axis_index("c")

        def body(o, carry):
            def c_of(t):
                return w + _NW * (o * chb + t)

            pltpu.sync_copy(idx_hbm.at[c_of(0)], ia)
            h_i = pltpu.async_copy(idx_hbm.at[c_of(1)], ib, isem)
            h_g = pltpu.async_copy(table_hbm.at[ia], bufs[0], gsem)
            for t in range(chb):
                p = t % 2
                h_g.wait()
                if t < chb - 1:
                    h_i.wait()
                    h_g = pltpu.async_copy(
                        table_hbm.at[ibufs[(t + 1) % 2]], bufs[1 - p], gsem
                    )
                    if t < chb - 2:
                        h_i = pltpu.async_copy(idx_hbm.at[c_of(t + 2)], ibufs[p], isem)
                pltpu.sync_copy(bufs[p], out_hbm.at[c_of(t)])
            return carry

        lax.fori_loop(0, nouter, body, 0)

    return k(idx2d, table)


def _tc_linear(fflat, Wt, b2, E, tile):
    nb = E // tile
    out_f = Wt.shape[2]
    F = Wt.shape[1]

    def body(f0, f1, f2, f3, f4, wref, bref, oref):
        a0, a1, a2, a3, a4 = f0[...], f1[...], f2[...], f3[...], f4[...]
        cs = (a0, a1 + a3, a2 + a4, jnp.abs(a1 - a3), jnp.abs(a2 - a4))
        acc = jnp.broadcast_to(bref[...].astype(jnp.float32), (tile, out_f))
        for i, c in enumerate(cs):
            acc += jnp.dot(
                c.astype(jnp.bfloat16), wref[i], preferred_element_type=jnp.float32
            )
        oref[...] = acc

    fspec = [
        pl.BlockSpec((tile, F), lambda i, k=k: (k * nb + i, 0)) for k in range(5)
    ]
    return pl.pallas_call(
        body,
        grid=(nb,),
        in_specs=fspec
        + [
            pl.BlockSpec((5, F, out_f), lambda i: (0, 0, 0)),
            pl.BlockSpec((1, out_f), lambda i: (0, 0)),
        ],
        out_specs=pl.BlockSpec((tile, out_f), lambda i: (i, 0)),
        out_shape=jax.ShapeDtypeStruct((E, out_f), jnp.float32),
    )(fflat, fflat, fflat, fflat, fflat, Wt, b2)


def kernel(x, edgemat, W, b):
    _, F, E, _ = x.shape
    K = edgemat.shape[2]
    out_f = W.shape[0]
    xt = jnp.transpose(x[0, :, :, 0])
    idx2d = jnp.transpose(edgemat[0]).reshape(-1, 128)
    nrows = idx2d.shape[0]
    align = _NW * 2 * _SUP
    npad = (nrows + align - 1) // align * align
    idx_pad = jnp.concatenate(
        [idx2d, jnp.zeros((npad - nrows, 128), jnp.int32)], axis=0
    )
    fgath = _sc_gather(idx_pad, xt)
    Wt = jnp.transpose(W.reshape(out_f, K, F), (1, 2, 0)).astype(jnp.bfloat16)
    y = _tc_linear(fgath.reshape(npad * 128, F), Wt, b.reshape(1, -1), E, tile=512)
    return jnp.transpose(y)[None, :, :, None]
